# baseline (speedup 1.0000x reference)
import functools

import jax
import jax.numpy as jnp
from jax import lax
from jax.experimental import pallas as pl
from jax.experimental.pallas import tpu as pltpu
from jax.experimental.pallas import tpu_sc as plsc

B = 16384
D = 32
_LANES = 128
_PACK = _LANES // D
_VROWS = 1000000 // _PACK

_info = plsc.get_sparse_core_info()
_NC, _NS = _info.num_cores, _info.num_subcores
_NW = _NC * _NS
_BPW = B // _NW

_mesh = plsc.VectorSubcoreMesh(core_axis_name="c", subcore_axis_name="s")


@functools.partial(
    pl.kernel,
    mesh=_mesh,
    out_type=jax.ShapeDtypeStruct((B, _LANES), jnp.float32),
    scratch_types=[
        pltpu.VMEM((_BPW,), jnp.int32),
        pltpu.VMEM((_BPW,), jnp.int32),
        pltpu.VMEM((_BPW, _LANES), jnp.float32),
        pltpu.SemaphoreType.DMA,
        pltpu.SemaphoreType.DMA,
    ],
    compiler_params=pltpu.CompilerParams(use_tc_tiling_on_sc=True),
)
def _gather_packed(idx_hbm, tab4_hbm, out4_hbm, idx_v, q_v, rows_v, sem_i, sem):
    wid = lax.axis_index("s") * _NC + lax.axis_index("c")
    base = wid * _BPW
    pltpu.async_copy(idx_hbm.at[pl.ds(base, _BPW)], idx_v, sem_i).wait()

    def mkq(t):
        q_v[pl.ds(t * 16, 16)] = lax.shift_right_logical(
            idx_v[pl.ds(t * 16, 16)], 2
        )

    pl.loop(0, _BPW // 16)(mkq)
    pltpu.async_copy(tab4_hbm.at[q_v], rows_v, sem).wait()
    pltpu.sync_copy(rows_v, out4_hbm.at[pl.ds(base, _BPW)])


@functools.partial(
    pl.kernel,
    mesh=_mesh,
    out_type=jax.ShapeDtypeStruct((B, D), jnp.float32),
    scratch_types=[
        pltpu.VMEM((_BPW,), jnp.int32),
        pltpu.VMEM((_BPW,), jnp.int32),
        pltpu.VMEM((_BPW, D), jnp.float32),
        pltpu.SemaphoreType.DMA,
        pltpu.SemaphoreType.DMA,
    ],
    compiler_params=pltpu.CompilerParams(use_tc_tiling_on_sc=False),
)
def _select_subrows(idx_hbm, out4v_hbm, out_hbm, idx_v, pos_v, rows_v, sem_i, sem):
    wid = lax.axis_index("s") * _NC + lax.axis_index("c")
    base = wid * _BPW
    pltpu.async_copy(idx_hbm.at[pl.ds(base, _BPW)], idx_v, sem_i).wait()

    lane16 = lax.iota(jnp.int32, 16)

    def mkpos(t):
        b16 = lane16 + (base + t * 16)
        pos_v[pl.ds(t * 16, 16)] = b16 * _PACK + (
            idx_v[pl.ds(t * 16, 16)] & (_PACK - 1)
        )

    pl.loop(0, _BPW // 16)(mkpos)
    pltpu.async_copy(out4v_hbm.at[pos_v], rows_v, sem).wait()
    pltpu.sync_copy(rows_v, out_hbm.at[pl.ds(base, _BPW)])


def kernel(cls_idx, table):
    idx32 = cls_idx.astype(jnp.int32)
    tab4 = table.reshape(_VROWS, _LANES)
    out4 = _gather_packed(idx32, tab4)
    out = _select_subrows(idx32, out4.reshape(B * _PACK, D))
    return out[:, None, :]


# speedup vs baseline: 1.0012x; 1.0012x over previous
import functools

import jax
import jax.numpy as jnp
from jax import lax
from jax.experimental import pallas as pl
from jax.experimental.pallas import tpu as pltpu
from jax.experimental.pallas import tpu_sc as plsc

B = 16384
D = 32
_LANES = 128
_PACK = _LANES // D
_VROWS = 1000000 // _PACK

_info = plsc.get_sparse_core_info()
_NC, _NS = _info.num_cores, _info.num_subcores
_NW = _NC * _NS
_BPW = B // _NW

_mesh = plsc.VectorSubcoreMesh(core_axis_name="c", subcore_axis_name="s")


@functools.partial(
    pl.kernel,
    mesh=_mesh,
    out_type=jax.ShapeDtypeStruct((B, _LANES), jnp.float32),
    scratch_types=[
        pltpu.VMEM((_BPW,), jnp.int32),
        pltpu.VMEM((_BPW,), jnp.int32),
        pltpu.VMEM((_BPW, _LANES), jnp.float32),
        pltpu.SemaphoreType.DMA,
        pltpu.SemaphoreType.DMA,
    ],
    compiler_params=pltpu.CompilerParams(use_tc_tiling_on_sc=True),
)
def _gather_packed(idx_hbm, tab4_hbm, out4_hbm, idx_v, q_v, rows_v, sem_i, sem):
    wid = lax.axis_index("s") * _NC + lax.axis_index("c")
    base = wid * _BPW
    pltpu.async_copy(idx_hbm.at[pl.ds(base, _BPW)], idx_v, sem_i).wait()

    def mkq(t):
        q_v[pl.ds(t * 16, 16)] = lax.shift_right_logical(
            idx_v[pl.ds(t * 16, 16)], 2
        )

    pl.loop(0, _BPW // 16)(mkq)
    pltpu.async_copy(tab4_hbm.at[q_v], rows_v, sem).wait()
    pltpu.sync_copy(rows_v, out4_hbm.at[pl.ds(base, _BPW)])


@functools.partial(
    pl.kernel,
    mesh=_mesh,
    out_type=jax.ShapeDtypeStruct((B, D), jnp.float32),
    scratch_types=[
        pltpu.VMEM((_BPW,), jnp.int32),
        pltpu.VMEM((_BPW,), jnp.int32),
        pltpu.VMEM((_BPW, D), jnp.float32),
        pltpu.SemaphoreType.DMA,
        pltpu.SemaphoreType.DMA,
    ],
    compiler_params=pltpu.CompilerParams(use_tc_tiling_on_sc=False),
)
def _select_subrows(idx_hbm, out4v_hbm, out_hbm, idx_v, pos_v, rows_v, sem_i, sem):
    wid = lax.axis_index("s") * _NC + lax.axis_index("c")
    base = wid * _BPW
    pltpu.async_copy(idx_hbm.at[pl.ds(base, _BPW)], idx_v, sem_i).wait()

    lane16 = lax.iota(jnp.int32, 16)

    def mkpos(t):
        b16 = lane16 + (base + t * 16)
        pos_v[pl.ds(t * 16, 16)] = b16 * _PACK + (
            idx_v[pl.ds(t * 16, 16)] & (_PACK - 1)
        )

    pl.loop(0, _BPW // 16)(mkpos)
    pltpu.async_copy(out4v_hbm.at[pos_v], rows_v, sem).wait()
    pltpu.sync_copy(rows_v, out_hbm.at[pl.ds(base, _BPW)])


def kernel(cls_idx, table):
    idx32 = cls_idx.astype(jnp.int32)
    tab4 = lax.optimization_barrier(table.reshape(_VROWS, _LANES))
    out4 = _gather_packed(idx32, tab4)
    out = _select_subrows(idx32, out4.reshape(B * _PACK, D))
    return out[:, None, :]


# speedup vs baseline: 1.0227x; 1.0215x over previous
import functools

import jax
import jax.numpy as jnp
from jax import lax
from jax.experimental import pallas as pl
from jax.experimental.pallas import tpu as pltpu
from jax.experimental.pallas import tpu_sc as plsc

B = 16384
D = 32

_info = plsc.get_sparse_core_info()
_NC, _NS = _info.num_cores, _info.num_subcores
_NW = _NC * _NS
_BPW = B // _NW

_mesh = plsc.VectorSubcoreMesh(core_axis_name="c", subcore_axis_name="s")


@functools.partial(
    pl.kernel,
    mesh=_mesh,
    out_type=jax.ShapeDtypeStruct((B, D), jnp.float32),
    scratch_types=[
        pltpu.VMEM((_BPW,), jnp.int32),
        pltpu.VMEM((_BPW, D), jnp.float32),
        pltpu.SemaphoreType.DMA,
    ],
    compiler_params=pltpu.CompilerParams(use_tc_tiling_on_sc=False),
)
def _embed_gather(idx_hbm, table_hbm, out_hbm, idx_v, rows_v, sem):
    wid = lax.axis_index("s") * _NC + lax.axis_index("c")
    base = wid * _BPW
    pltpu.sync_copy(idx_hbm.at[pl.ds(base, _BPW)], idx_v)
    pltpu.async_copy(table_hbm.at[idx_v], rows_v, sem).wait()
    pltpu.sync_copy(rows_v, out_hbm.at[pl.ds(base, _BPW)])


def kernel(cls_idx, table):
    out = _embed_gather(cls_idx.astype(jnp.int32), table)
    return out[:, None, :]
